# baseline (speedup 1.0000x reference)
import functools

import jax
import jax.numpy as jnp
from jax import lax
from jax.experimental import pallas as pl
from jax.experimental.pallas import tpu as pltpu

T = 128
E = 64
D = 768
F = 3072
FT = 512
NFT = F // FT
RT = 8
NRT = T // RT

_INV_SQRT2 = 0.7071067811865476


def _router_body(x_ref, wr_ref, br_ref, tab_ref, ptab_ref, cnt_ref):
    xx = x_ref[...]
    logits = jnp.dot(xx, wr_ref[...], preferred_element_type=jnp.float32)
    logits = logits + br_ref[...]
    m = jnp.max(logits, axis=1, keepdims=True)
    p = jnp.exp(logits - m)
    p = p / jnp.sum(p, axis=1, keepdims=True)

    cols = lax.broadcasted_iota(jnp.int32, (T, E), 1)
    m1 = jnp.max(p, axis=1, keepdims=True)
    i1 = jnp.min(jnp.where(p == m1, cols, E), axis=1, keepdims=True)
    pm = jnp.where(cols == i1, -1.0, p)
    m2 = jnp.max(pm, axis=1, keepdims=True)
    i2 = jnp.min(jnp.where(pm == m2, cols, E), axis=1, keepdims=True)
    s = m1 + m2
    w1 = m1 / s
    w2 = m2 / s

    oh1 = (cols == i1).astype(jnp.float32)
    oh2 = (cols == i2).astype(jnp.float32)

    rows_t = lax.broadcasted_iota(jnp.int32, (T, T), 0)
    cols_t = lax.broadcasted_iota(jnp.int32, (T, T), 1)
    ltri = (rows_t > cols_t).astype(jnp.float32)
    p1 = jnp.dot(ltri, oh1, preferred_element_type=jnp.float32)
    p2 = jnp.dot(ltri, oh2, preferred_element_type=jnp.float32)
    c1 = jnp.sum(oh1, axis=0, keepdims=True)

    rank1 = jnp.sum(p1 * oh1, axis=1, keepdims=True)
    rank2 = jnp.sum((p2 + c1) * oh2, axis=1, keepdims=True)

    slots = lax.broadcasted_iota(jnp.int32, (T, T), 1).astype(jnp.float32)
    s1 = (rank1 == slots).astype(jnp.float32)
    s2 = (rank2 == slots).astype(jnp.float32)
    tok = lax.broadcasted_iota(jnp.int32, (T, 1), 0).astype(jnp.float32)

    dn = (((0,), (0,)), ((), ()))
    tab = lax.dot_general(oh1, s1 * tok, dn, preferred_element_type=jnp.float32)
    tab = tab + lax.dot_general(oh2, s2 * tok, dn, preferred_element_type=jnp.float32)
    ptab = lax.dot_general(oh1, s1 * w1, dn, preferred_element_type=jnp.float32)
    ptab = ptab + lax.dot_general(oh2, s2 * w2, dn, preferred_element_type=jnp.float32)

    tab_ref[...] = tab
    ptab_ref[...] = ptab
    cnt_ref[...] = c1 + jnp.sum(oh2, axis=0, keepdims=True)


def _ffn_body(cnt_ref, x_ref, w1_ref, w2_ref, b1_ref, b2_ref, tab_ref,
              ptab_ref, out_ref):
    e = pl.program_id(0)
    f = pl.program_id(1)

    @pl.when((e == 0) & (f == 0))
    def _():
        out_ref[...] = jnp.zeros_like(out_ref)

    cnt = cnt_ref[e]
    xv = x_ref[...]
    onehot_e = (lax.broadcasted_iota(jnp.int32, (1, E), 1) == e).astype(
        jnp.float32)
    b1c = jnp.dot(onehot_e, b1_ref[...], preferred_element_type=jnp.float32)
    b2r = jnp.dot(onehot_e, b2_ref[...], preferred_element_type=jnp.float32)
    w1 = w1_ref[0]
    w2 = w2_ref[0]
    ito = lax.broadcasted_iota(jnp.int32, (RT, T), 1).astype(jnp.float32)
    b2scale = jnp.where(f == 0, 1.0, 0.0)

    for r in range(NRT):
        @pl.when(cnt > r * RT)
        def _(r=r):
            col = tab_ref[0, :, r:r + 1]
            pcol = ptab_ref[0, :, r:r + 1]
            g = (col == ito).astype(jnp.float32)
            xg = jnp.dot(g, xv, preferred_element_type=jnp.float32)
            h = jnp.dot(xg, w1, preferred_element_type=jnp.float32) + b1c
            h = 0.5 * h * (1.0 + lax.erf(h * _INV_SQRT2))
            part = jnp.dot(h, w2, preferred_element_type=jnp.float32)
            part = part + b2scale * b2r
            contrib = pcol * part
            out_ref[...] += lax.dot_general(
                g, contrib, (((0,), (0,)), ((), ())),
                preferred_element_type=jnp.float32)


@jax.jit
def kernel(x, Wr, br, W1, b1, W2, b2):
    B, S, _ = x.shape
    x2 = x.reshape(T, D)

    tab, ptab, cnt = pl.pallas_call(
        _router_body,
        out_shape=[
            jax.ShapeDtypeStruct((E, T), jnp.float32),
            jax.ShapeDtypeStruct((E, T), jnp.float32),
            jax.ShapeDtypeStruct((1, E), jnp.float32),
        ],
    )(x2, Wr, br.reshape(1, E))

    tab3 = tab.reshape(E, NRT, RT).swapaxes(1, 2)
    ptab3 = ptab.reshape(E, NRT, RT).swapaxes(1, 2)
    cnt_i = cnt.reshape(E).astype(jnp.int32)

    grid = (E, NFT)
    out = pl.pallas_call(
        _ffn_body,
        grid=grid,
        in_specs=[
            pl.BlockSpec(memory_space=pltpu.SMEM),
            pl.BlockSpec((T, D), lambda e, f: (0, 0)),
            pl.BlockSpec((1, D, FT), lambda e, f: (e, 0, f)),
            pl.BlockSpec((1, FT, D), lambda e, f: (e, f, 0)),
            pl.BlockSpec((E, FT), lambda e, f: (0, f)),
            pl.BlockSpec((E, D), lambda e, f: (0, 0)),
            pl.BlockSpec((1, RT, NRT), lambda e, f: (e, 0, 0)),
            pl.BlockSpec((1, RT, NRT), lambda e, f: (e, 0, 0)),
        ],
        out_specs=pl.BlockSpec((T, D), lambda e, f: (0, 0)),
        out_shape=jax.ShapeDtypeStruct((T, D), jnp.float32),
    )(cnt_i, x2, W1, W2, b1, b2, tab3, ptab3)

    return out.reshape(B, S, D)


# speedup vs baseline: 1.6736x; 1.6736x over previous
import jax
import jax.numpy as jnp
from jax import lax
from jax.experimental import pallas as pl
from jax.experimental.pallas import tpu as pltpu

T = 128
E = 64
D = 768
F = 3072
RT = 8
NRT = T // RT

_INV_SQRT2 = 0.7071067811865476


def _router_body(x_ref, wr_ref, br_ref, tab_ref, ptab_ref, cnt_ref):
    xx = x_ref[...]
    logits = jnp.dot(xx, wr_ref[...], preferred_element_type=jnp.float32)
    logits = logits + br_ref[...]
    m = jnp.max(logits, axis=1, keepdims=True)
    p = jnp.exp(logits - m)
    p = p / jnp.sum(p, axis=1, keepdims=True)

    cols = lax.broadcasted_iota(jnp.int32, (T, E), 1)
    m1 = jnp.max(p, axis=1, keepdims=True)
    i1 = jnp.min(jnp.where(p == m1, cols, E), axis=1, keepdims=True)
    pm = jnp.where(cols == i1, -1.0, p)
    m2 = jnp.max(pm, axis=1, keepdims=True)
    i2 = jnp.min(jnp.where(pm == m2, cols, E), axis=1, keepdims=True)
    s = m1 + m2
    w1 = m1 / s
    w2 = m2 / s

    oh1 = (cols == i1).astype(jnp.float32)
    oh2 = (cols == i2).astype(jnp.float32)

    rows_t = lax.broadcasted_iota(jnp.int32, (T, T), 0)
    cols_t = lax.broadcasted_iota(jnp.int32, (T, T), 1)
    ltri = (rows_t > cols_t).astype(jnp.float32)
    p1 = jnp.dot(ltri, oh1, preferred_element_type=jnp.float32)
    p2 = jnp.dot(ltri, oh2, preferred_element_type=jnp.float32)
    c1 = jnp.sum(oh1, axis=0, keepdims=True)

    rank1 = jnp.sum(p1 * oh1, axis=1, keepdims=True)
    rank2 = jnp.sum((p2 + c1) * oh2, axis=1, keepdims=True)

    def enc(q):
        fl = jnp.floor(q * 0.125)
        return (q - 8.0 * fl) * 16.0 + fl

    slots = lax.broadcasted_iota(jnp.int32, (T, T), 1).astype(jnp.float32)
    s1 = (enc(rank1) == slots).astype(jnp.float32)
    s2 = (enc(rank2) == slots).astype(jnp.float32)
    tok = lax.broadcasted_iota(jnp.int32, (T, 1), 0).astype(jnp.float32)

    dn = (((0,), (0,)), ((), ()))
    tab = lax.dot_general(oh1, s1 * tok, dn, preferred_element_type=jnp.float32)
    tab = tab + lax.dot_general(oh2, s2 * tok, dn, preferred_element_type=jnp.float32)
    ptab = lax.dot_general(oh1, s1 * w1, dn, preferred_element_type=jnp.float32)
    ptab = ptab + lax.dot_general(oh2, s2 * w2, dn, preferred_element_type=jnp.float32)

    tab_ref[...] = tab
    ptab_ref[...] = ptab
    cnt_ref[...] = c1 + jnp.sum(oh2, axis=0, keepdims=True)


def _ffn_body(cnt_ref, x_ref, w1_ref, w2_ref, b1_ref, b2_ref, tab_ref,
              ptab_ref, out_ref):
    e = pl.program_id(0)

    @pl.when(e == 0)
    def _():
        out_ref[...] = jnp.zeros_like(out_ref)

    cnt = cnt_ref[e]
    xv = x_ref[...].astype(jnp.bfloat16)
    onehot_e = (lax.broadcasted_iota(jnp.int32, (1, E), 1) == e).astype(
        jnp.float32)
    b1c = jnp.dot(onehot_e, b1_ref[...], preferred_element_type=jnp.float32)
    b2r = jnp.dot(onehot_e, b2_ref[...], preferred_element_type=jnp.float32)
    w1 = w1_ref[0].astype(jnp.bfloat16)
    w2 = w2_ref[0].astype(jnp.bfloat16)
    ito = lax.broadcasted_iota(jnp.int32, (RT, T), 1).astype(jnp.float32)

    for r in range(NRT):
        @pl.when(cnt > r * RT)
        def _(r=r):
            col = tab_ref[0, :, r:r + 1]
            pcol = ptab_ref[0, :, r:r + 1]
            g = (col == ito).astype(jnp.bfloat16)
            xg = jnp.dot(g, xv, preferred_element_type=jnp.float32)
            h = jnp.dot(xg.astype(jnp.bfloat16), w1,
                        preferred_element_type=jnp.float32) + b1c
            h = 0.5 * h * (1.0 + lax.erf(h * _INV_SQRT2))
            part = jnp.dot(h.astype(jnp.bfloat16), w2,
                           preferred_element_type=jnp.float32)
            part = part + b2r
            contrib = (pcol * part).astype(jnp.bfloat16)
            out_ref[...] += lax.dot_general(
                g, contrib, (((0,), (0,)), ((), ())),
                preferred_element_type=jnp.float32)


@jax.jit
def kernel(x, Wr, br, W1, b1, W2, b2):
    B, S, _ = x.shape
    x2 = x.reshape(T, D)

    tab, ptab, cnt = pl.pallas_call(
        _router_body,
        out_shape=[
            jax.ShapeDtypeStruct((E, T), jnp.float32),
            jax.ShapeDtypeStruct((E, T), jnp.float32),
            jax.ShapeDtypeStruct((1, E), jnp.float32),
        ],
    )(x2, Wr, br.reshape(1, E))

    tab3 = tab.reshape(E, RT, NRT)
    ptab3 = ptab.reshape(E, RT, NRT)
    cnt_i = cnt.reshape(E).astype(jnp.int32)

    out = pl.pallas_call(
        _ffn_body,
        grid=(E,),
        in_specs=[
            pl.BlockSpec(memory_space=pltpu.SMEM),
            pl.BlockSpec((T, D), lambda e: (0, 0)),
            pl.BlockSpec((1, D, F), lambda e: (e, 0, 0)),
            pl.BlockSpec((1, F, D), lambda e: (e, 0, 0)),
            pl.BlockSpec((E, F), lambda e: (0, 0)),
            pl.BlockSpec((E, D), lambda e: (0, 0)),
            pl.BlockSpec((1, RT, NRT), lambda e: (e, 0, 0)),
            pl.BlockSpec((1, RT, NRT), lambda e: (e, 0, 0)),
        ],
        out_specs=pl.BlockSpec((T, D), lambda e: (0, 0)),
        out_shape=jax.ShapeDtypeStruct((T, D), jnp.float32),
    )(cnt_i, x2, W1, W2, b1, b2, tab3, ptab3)

    return out.reshape(B, S, D)


# speedup vs baseline: 1.7145x; 1.0244x over previous
import jax
import jax.numpy as jnp
from jax import lax
from jax.experimental import pallas as pl
from jax.experimental.pallas import tpu as pltpu

T = 128
E = 64
D = 768
F = 3072
RT = 8
NRT = T // RT

_INV_SQRT2 = 0.7071067811865476
_DN0 = (((0,), (0,)), ((), ()))


def _router_body(x_ref, wr_ref, br_ref, tab_ref, ptab_ref, cnt_ref,
                 order_ref, nhit_ref):
    xx = x_ref[...]
    logits = jnp.dot(xx, wr_ref[...], preferred_element_type=jnp.float32)
    logits = logits + br_ref[...]
    m = jnp.max(logits, axis=1, keepdims=True)
    p = jnp.exp(logits - m)
    p = p / jnp.sum(p, axis=1, keepdims=True)

    cols = lax.broadcasted_iota(jnp.int32, (T, E), 1)
    m1 = jnp.max(p, axis=1, keepdims=True)
    i1 = jnp.min(jnp.where(p == m1, cols, E), axis=1, keepdims=True)
    pm = jnp.where(cols == i1, -1.0, p)
    m2 = jnp.max(pm, axis=1, keepdims=True)
    i2 = jnp.min(jnp.where(pm == m2, cols, E), axis=1, keepdims=True)
    s = m1 + m2
    w1 = m1 / s
    w2 = m2 / s

    oh1 = (cols == i1).astype(jnp.float32)
    oh2 = (cols == i2).astype(jnp.float32)

    rows_t = lax.broadcasted_iota(jnp.int32, (T, T), 0)
    cols_t = lax.broadcasted_iota(jnp.int32, (T, T), 1)
    ltri = (rows_t > cols_t).astype(jnp.float32)
    p1 = jnp.dot(ltri, oh1, preferred_element_type=jnp.float32)
    p2 = jnp.dot(ltri, oh2, preferred_element_type=jnp.float32)
    c1 = jnp.sum(oh1, axis=0, keepdims=True)

    rank1 = jnp.sum(p1 * oh1, axis=1, keepdims=True)
    rank2 = jnp.sum((p2 + c1) * oh2, axis=1, keepdims=True)

    def enc(q):
        fl = jnp.floor(q * 0.125)
        return (q - 8.0 * fl) * 16.0 + fl

    slots = lax.broadcasted_iota(jnp.int32, (T, T), 1).astype(jnp.float32)
    s1 = (enc(rank1) == slots).astype(jnp.float32)
    s2 = (enc(rank2) == slots).astype(jnp.float32)
    tok = lax.broadcasted_iota(jnp.int32, (T, 1), 0).astype(jnp.float32)

    tab = lax.dot_general(oh1, s1 * tok, _DN0, preferred_element_type=jnp.float32)
    tab = tab + lax.dot_general(oh2, s2 * tok, _DN0, preferred_element_type=jnp.float32)
    ptab = lax.dot_general(oh1, s1 * w1, _DN0, preferred_element_type=jnp.float32)
    ptab = ptab + lax.dot_general(oh2, s2 * w2, _DN0, preferred_element_type=jnp.float32)

    cnt = c1 + jnp.sum(oh2, axis=0, keepdims=True)

    ones_t = jnp.ones((T, 1), jnp.float32)
    cnt_col = lax.dot_general(oh1 + oh2, ones_t, _DN0,
                              preferred_element_type=jnp.float32)
    hit_col = (cnt_col > 0.0).astype(jnp.float32)
    er = lax.broadcasted_iota(jnp.int32, (E, E), 0)
    ec = lax.broadcasted_iota(jnp.int32, (E, E), 1)
    ltriE = (ec < er).astype(jnp.float32)
    pos_col = jnp.dot(ltriE, hit_col, preferred_element_type=jnp.float32)
    p_iotaE = lax.broadcasted_iota(jnp.int32, (E, E), 1).astype(jnp.float32)
    mm = jnp.where(pos_col == p_iotaE, hit_col, 0.0)
    e_col = lax.broadcasted_iota(jnp.int32, (E, 1), 0).astype(jnp.float32)
    order = lax.dot_general(e_col, mm, _DN0,
                            preferred_element_type=jnp.float32)
    nhit = jnp.sum(hit_col, axis=0, keepdims=True)
    last_hit = jnp.max(e_col * hit_col, axis=0, keepdims=True)
    prow = lax.broadcasted_iota(jnp.int32, (1, E), 1).astype(jnp.float32)
    order = order + jnp.where(prow >= nhit, last_hit, 0.0)

    tab_ref[...] = tab
    ptab_ref[...] = ptab
    cnt_ref[...] = cnt.astype(jnp.int32)
    order_ref[...] = order.astype(jnp.int32)
    nhit_ref[...] = nhit.astype(jnp.int32)


def _ffn_body(order_ref, cnt_ref, nhit_ref, x_ref, w1_ref, w2_ref, b1_ref,
              b2_ref, tab_ref, ptab_ref, out_ref):
    i = pl.program_id(0)

    @pl.when(i == 0)
    def _():
        out_ref[...] = jnp.zeros_like(out_ref)

    eo = order_ref[0, i]
    live = i < nhit_ref[0, 0]
    cnt = cnt_ref[0, eo]
    xv = x_ref[...].astype(jnp.bfloat16)
    onehot_e = (lax.broadcasted_iota(jnp.int32, (1, E), 1) == eo).astype(
        jnp.float32)
    b1c = jnp.dot(onehot_e, b1_ref[...], preferred_element_type=jnp.float32)
    b2r = jnp.dot(onehot_e, b2_ref[...], preferred_element_type=jnp.float32)
    w1 = w1_ref[0].astype(jnp.bfloat16)
    w2 = w2_ref[0].astype(jnp.bfloat16)
    ito = lax.broadcasted_iota(jnp.int32, (RT, T), 1).astype(jnp.float32)

    for r in range(NRT):
        @pl.when(live & (cnt > r * RT))
        def _(r=r):
            col = tab_ref[0, :, r:r + 1]
            pcol = ptab_ref[0, :, r:r + 1]
            g = (col == ito).astype(jnp.bfloat16)
            xg = jnp.dot(g, xv, preferred_element_type=jnp.float32)
            h = jnp.dot(xg.astype(jnp.bfloat16), w1,
                        preferred_element_type=jnp.float32) + b1c
            h = 0.5 * h * (1.0 + lax.erf(h * _INV_SQRT2))
            part = jnp.dot(h.astype(jnp.bfloat16), w2,
                           preferred_element_type=jnp.float32)
            part = part + b2r
            contrib = (pcol * part).astype(jnp.bfloat16)
            out_ref[...] += lax.dot_general(
                g, contrib, _DN0, preferred_element_type=jnp.float32)


@jax.jit
def kernel(x, Wr, br, W1, b1, W2, b2):
    B, S, _ = x.shape
    x2 = x.reshape(T, D)

    tab, ptab, cnt, order, nhit = pl.pallas_call(
        _router_body,
        out_shape=[
            jax.ShapeDtypeStruct((E, T), jnp.float32),
            jax.ShapeDtypeStruct((E, T), jnp.float32),
            jax.ShapeDtypeStruct((1, E), jnp.int32),
            jax.ShapeDtypeStruct((1, E), jnp.int32),
            jax.ShapeDtypeStruct((1, 1), jnp.int32),
        ],
    )(x2, Wr, br.reshape(1, E))

    tab3 = tab.reshape(E, RT, NRT)
    ptab3 = ptab.reshape(E, RT, NRT)

    grid_spec = pltpu.PrefetchScalarGridSpec(
        num_scalar_prefetch=3,
        grid=(E,),
        in_specs=[
            pl.BlockSpec((T, D), lambda i, o, c, n: (0, 0)),
            pl.BlockSpec((1, D, F), lambda i, o, c, n: (o[0, i], 0, 0)),
            pl.BlockSpec((1, F, D), lambda i, o, c, n: (o[0, i], 0, 0)),
            pl.BlockSpec((E, F), lambda i, o, c, n: (0, 0)),
            pl.BlockSpec((E, D), lambda i, o, c, n: (0, 0)),
            pl.BlockSpec((1, RT, NRT), lambda i, o, c, n: (o[0, i], 0, 0)),
            pl.BlockSpec((1, RT, NRT), lambda i, o, c, n: (o[0, i], 0, 0)),
        ],
        out_specs=pl.BlockSpec((T, D), lambda i, o, c, n: (0, 0)),
    )
    out = pl.pallas_call(
        _ffn_body,
        grid_spec=grid_spec,
        out_shape=jax.ShapeDtypeStruct((T, D), jnp.float32),
    )(order, cnt, nhit, x2, W1, W2, b1, b2, tab3, ptab3)

    return out.reshape(B, S, D)
